# baseline (speedup 1.0000x reference)
import jax
import jax.numpy as jnp
from jax import lax
from jax.experimental import pallas as pl
from jax.experimental.pallas import tpu as pltpu
from jax.experimental.pallas import tpu_sc as plsc

N_USERS = 20000
N_ITEMS = 30000
N_NODES = N_USERS + N_ITEMS
N_EDGES = 800000
EMBED_DIM = 64
N_LAYERS = 3

NC = 2
NS = 16
L = 16

NPAD = 51200
DH = EMBED_DIM // NC

ZR = NPAD // NS
QR = 80
NQ = ZR // QR
NBUF = 4

E_TILE = N_EDGES // NS
K = 80
G = 25
BLK = K * G
NB = E_TILE // BLK


def _layer_body(p_prev, acc_prev, rows3, cols3, vals3,
                p_next, acc_next,
                table,
                gbufa, gbufb, gbufc, gbufd, gbufe, gbuff, colv3, rowv3, valv3,
                gsa, gsb, gsc, gsd, gse, gsf,
                ssa, ssb, ssc, ssd, sse, ssf, ri, ci, vi):
    c = lax.axis_index("c")
    s = lax.axis_index("s")

    gbufs = (gbufa, gbufb, gbufc, gbufd, gbufe, gbuff)
    gsems = (gsa, gsb, gsc, gsd, gse, gsf)
    ssems = (ssa, ssb, ssc, ssd, sse, ssf)

    def _zfill(r, carry):
        for j in range(DH // L):
            gbufa[r, pl.ds(j * L, L)] = jnp.zeros((L,), jnp.float32)
        return carry
    lax.fori_loop(0, QR, _zfill, 0)
    zd = [pltpu.async_copy(gbufa, table.at[pl.ds(s * ZR + q * QR, QR)], gsa)
          for q in range(NQ)]
    for d in zd:
        d.wait()
    plsc.subcore_barrier()

    psrc = p_prev.at[c]

    pltpu.async_copy(rows3.at[s * NB], rowv3.at[0], ri)
    pltpu.async_copy(cols3.at[s * NB], colv3.at[0], ci)
    pltpu.async_copy(vals3.at[s * NB], valv3.at[0], vi)

    def _block(b, carry):
        par = b % 2
        rowv2 = rowv3.at[par]
        colv2 = colv3.at[par]
        valv2 = valv3.at[par]
        pltpu.make_async_copy(rows3.at[s * NB], rowv2, ri).wait()
        pltpu.make_async_copy(cols3.at[s * NB], colv2, ci).wait()
        pltpu.make_async_copy(vals3.at[s * NB], valv2, vi).wait()

        @pl.when(b + 1 < NB)
        def _prefetch():
            nxt = (b + 1) % 2
            pltpu.async_copy(rows3.at[s * NB + b + 1], rowv3.at[nxt], ri)
            pltpu.async_copy(cols3.at[s * NB + b + 1], colv3.at[nxt], ci)
            pltpu.async_copy(vals3.at[s * NB + b + 1], valv3.at[nxt], vi)

        def _mul(g, buf):
            @plsc.parallel_loop(0, K // L)
            def _mq(q):
                vv16 = valv2[g, pl.ds(q * L, L)]
                for i in range(L):
                    vv = jnp.full((L,), vv16[i], jnp.float32)
                    for j in range(DH // L):
                        sl = (q * L + i, pl.ds(j * L, L))
                        buf[sl] = buf[sl] * vv

        gd = [None] * G
        sd = [None] * G
        for g in range(NBUF - 1):
            x = g % NBUF
            gd[g] = pltpu.async_copy(psrc.at[colv2.at[g]], gbufs[x], gsems[x])
        for gg in range(NBUF - 1, G + NBUF - 1):
            if gg < G:
                x = gg % NBUF
                if gg >= NBUF:
                    sd[gg - NBUF].wait()
                gd[gg] = pltpu.async_copy(psrc.at[colv2.at[gg]], gbufs[x],
                                          gsems[x])
            p = gg - (NBUF - 1)
            y = p % NBUF
            gd[p].wait()
            _mul(p, gbufs[y])
            sd[p] = pltpu.async_copy(gbufs[y], table.at[rowv2.at[p]],
                                     ssems[y], add=True)
        for p in range(G - NBUF, G):
            if p >= 0:
                sd[p].wait()
        return carry
    lax.fori_loop(0, NB, _block, 0)
    plsc.subcore_barrier()

    pd = pltpu.async_copy(table.at[pl.ds(s * ZR, ZR)],
                          p_next.at[c].at[pl.ds(s * ZR, ZR)], gse)
    tb = (gbufa, gbufb)
    ab = (gbufc, gbufd)
    st = (gsa, gsb)
    sa = (gsc, gsd)
    sw = (ssa, ssb)
    td = [None] * NQ
    ad = [None] * NQ
    wd = [None] * NQ
    for i in range(NQ + 1):
        x = i % 2
        if i < NQ:
            tro = s * ZR + i * QR
            if i >= 2:
                wd[i - 2].wait()
            td[i] = pltpu.async_copy(table.at[pl.ds(tro, QR)], tb[x], st[x])
            ad[i] = pltpu.async_copy(acc_prev.at[c].at[pl.ds(tro, QR)],
                                     ab[x], sa[x])
        if i >= 1:
            y = (i - 1) % 2
            td[i - 1].wait()
            ad[i - 1].wait()

            def _acc(r, carry):
                for j in range(DH // L):
                    tb[y][r, pl.ds(j * L, L)] = (tb[y][r, pl.ds(j * L, L)]
                                                 + ab[y][r, pl.ds(j * L, L)])
                return carry
            lax.fori_loop(0, QR, _acc, 0)
            wd[i - 1] = pltpu.async_copy(
                tb[y], acc_next.at[c].at[pl.ds(s * ZR + (i - 1) * QR, QR)],
                sw[y])
    wd[NQ - 2].wait()
    wd[NQ - 1].wait()
    pd.wait()


_layer = pl.kernel(
    _layer_body,
    out_type=(
        jax.ShapeDtypeStruct((NC, NPAD, DH), jnp.float32),
        jax.ShapeDtypeStruct((NC, NPAD, DH), jnp.float32),
    ),
    mesh=plsc.VectorSubcoreMesh(core_axis_name="c", subcore_axis_name="s",
                                num_cores=NC, num_subcores=NS),
    compiler_params=pltpu.CompilerParams(use_tc_tiling_on_sc=False),
    scratch_types=[
        pltpu.VMEM_SHARED((NPAD, DH), jnp.float32),
        pltpu.VMEM((K, DH), jnp.float32),
        pltpu.VMEM((K, DH), jnp.float32),
        pltpu.VMEM((K, DH), jnp.float32),
        pltpu.VMEM((K, DH), jnp.float32),
        pltpu.VMEM((K, DH), jnp.float32),
        pltpu.VMEM((K, DH), jnp.float32),
        pltpu.VMEM((2, G, K), jnp.int32),
        pltpu.VMEM((2, G, K), jnp.int32),
        pltpu.VMEM((2, G, K), jnp.float32),
        pltpu.SemaphoreType.DMA,
        pltpu.SemaphoreType.DMA,
        pltpu.SemaphoreType.DMA,
        pltpu.SemaphoreType.DMA,
        pltpu.SemaphoreType.DMA,
        pltpu.SemaphoreType.DMA,
        pltpu.SemaphoreType.DMA,
        pltpu.SemaphoreType.DMA,
        pltpu.SemaphoreType.DMA,
        pltpu.SemaphoreType.DMA,
        pltpu.SemaphoreType.DMA,
        pltpu.SemaphoreType.DMA,
        pltpu.SemaphoreType.DMA,
        pltpu.SemaphoreType.DMA,
        pltpu.SemaphoreType.DMA,
    ],
)


def kernel(user_preference, item_preference, edge_values, edge_index):
    p0 = jnp.concatenate([user_preference, item_preference], axis=0)
    p0 = jnp.pad(p0, ((0, NPAD - N_NODES), (0, 0)))
    p = jnp.stack([p0[:, :DH], p0[:, DH:]])
    acc = p
    rows3 = edge_index[0].astype(jnp.int32).reshape(NS * NB, G, K)
    cols3 = edge_index[1].astype(jnp.int32).reshape(NS * NB, G, K)
    vals3 = edge_values.astype(jnp.float32).reshape(NS * NB, G, K)
    for _ in range(N_LAYERS):
        p, acc = _layer(p, acc, rows3, cols3, vals3)
    full = jnp.concatenate([acc[0], acc[1]], axis=1)[:N_NODES]
    return (full[:N_USERS], full[N_USERS:])


# speedup vs baseline: 1.0251x; 1.0251x over previous
import jax
import jax.numpy as jnp
from jax import lax
from jax.experimental import pallas as pl
from jax.experimental.pallas import tpu as pltpu
from jax.experimental.pallas import tpu_sc as plsc

N_USERS = 20000
N_ITEMS = 30000
N_NODES = N_USERS + N_ITEMS
N_EDGES = 800000
EMBED_DIM = 64
N_LAYERS = 3

NC = 2
NS = 16
L = 16

NPAD = 51200
DH = EMBED_DIM // NC

ZR = NPAD // NS
QR = 80
NQ = ZR // QR
NBUF = 6

E_TILE = N_EDGES // NS
K = 80
G = 25
BLK = K * G
NB = E_TILE // BLK


def _layer_body(p_prev, acc_prev, rows3, cols3, vals3,
                p_next, acc_next,
                table,
                gbufa, gbufb, gbufc, gbufd, gbufe, gbuff, colv3, rowv3, valv3,
                gsa, gsb, gsc, gsd, gse, gsf,
                ssa, ssb, ssc, ssd, sse, ssf, ri, ci, vi):
    c = lax.axis_index("c")
    s = lax.axis_index("s")

    gbufs = (gbufa, gbufb, gbufc, gbufd, gbufe, gbuff)
    gsems = (gsa, gsb, gsc, gsd, gse, gsf)
    ssems = (ssa, ssb, ssc, ssd, sse, ssf)

    def _zfill(r, carry):
        for j in range(DH // L):
            gbufa[r, pl.ds(j * L, L)] = jnp.zeros((L,), jnp.float32)
        return carry
    lax.fori_loop(0, QR, _zfill, 0)
    zd = [pltpu.async_copy(gbufa, table.at[pl.ds(s * ZR + q * QR, QR)], gsa)
          for q in range(NQ)]
    for d in zd:
        d.wait()
    plsc.subcore_barrier()

    psrc = p_prev.at[c]

    pltpu.async_copy(rows3.at[s * NB], rowv3.at[0], ri)
    pltpu.async_copy(cols3.at[s * NB], colv3.at[0], ci)
    pltpu.async_copy(vals3.at[s * NB], valv3.at[0], vi)

    def _block(b, carry):
        par = b % 2
        rowv2 = rowv3.at[par]
        colv2 = colv3.at[par]
        valv2 = valv3.at[par]
        pltpu.make_async_copy(rows3.at[s * NB], rowv2, ri).wait()
        pltpu.make_async_copy(cols3.at[s * NB], colv2, ci).wait()
        pltpu.make_async_copy(vals3.at[s * NB], valv2, vi).wait()

        @pl.when(b + 1 < NB)
        def _prefetch():
            nxt = (b + 1) % 2
            pltpu.async_copy(rows3.at[s * NB + b + 1], rowv3.at[nxt], ri)
            pltpu.async_copy(cols3.at[s * NB + b + 1], colv3.at[nxt], ci)
            pltpu.async_copy(vals3.at[s * NB + b + 1], valv3.at[nxt], vi)

        def _mul(g, buf):
            @plsc.parallel_loop(0, K // L)
            def _mq(q):
                vv16 = valv2[g, pl.ds(q * L, L)]
                for i in range(L):
                    vv = jnp.full((L,), vv16[i], jnp.float32)
                    for j in range(DH // L):
                        sl = (q * L + i, pl.ds(j * L, L))
                        buf[sl] = buf[sl] * vv

        gd = [None] * G
        sd = [None] * G
        for g in range(NBUF - 1):
            x = g % NBUF
            gd[g] = pltpu.async_copy(psrc.at[colv2.at[g]], gbufs[x], gsems[x])
        for gg in range(NBUF - 1, G + NBUF - 1):
            if gg < G:
                x = gg % NBUF
                if gg >= NBUF:
                    sd[gg - NBUF].wait()
                gd[gg] = pltpu.async_copy(psrc.at[colv2.at[gg]], gbufs[x],
                                          gsems[x])
            p = gg - (NBUF - 1)
            y = p % NBUF
            gd[p].wait()
            _mul(p, gbufs[y])
            sd[p] = pltpu.async_copy(gbufs[y], table.at[rowv2.at[p]],
                                     ssems[y], add=True)
        for p in range(G - NBUF, G):
            if p >= 0:
                sd[p].wait()
        return carry
    lax.fori_loop(0, NB, _block, 0)
    plsc.subcore_barrier()

    pd = pltpu.async_copy(table.at[pl.ds(s * ZR, ZR)],
                          p_next.at[c].at[pl.ds(s * ZR, ZR)], gse)
    tb = (gbufa, gbufb)
    ab = (gbufc, gbufd)
    st = (gsa, gsb)
    sa = (gsc, gsd)
    sw = (ssa, ssb)
    td = [None] * NQ
    ad = [None] * NQ
    wd = [None] * NQ
    for i in range(NQ + 1):
        x = i % 2
        if i < NQ:
            tro = s * ZR + i * QR
            if i >= 2:
                wd[i - 2].wait()
            td[i] = pltpu.async_copy(table.at[pl.ds(tro, QR)], tb[x], st[x])
            ad[i] = pltpu.async_copy(acc_prev.at[c].at[pl.ds(tro, QR)],
                                     ab[x], sa[x])
        if i >= 1:
            y = (i - 1) % 2
            td[i - 1].wait()
            ad[i - 1].wait()

            def _acc(r, carry):
                for j in range(DH // L):
                    tb[y][r, pl.ds(j * L, L)] = (tb[y][r, pl.ds(j * L, L)]
                                                 + ab[y][r, pl.ds(j * L, L)])
                return carry
            lax.fori_loop(0, QR, _acc, 0)
            wd[i - 1] = pltpu.async_copy(
                tb[y], acc_next.at[c].at[pl.ds(s * ZR + (i - 1) * QR, QR)],
                sw[y])
    wd[NQ - 2].wait()
    wd[NQ - 1].wait()
    pd.wait()


_layer = pl.kernel(
    _layer_body,
    out_type=(
        jax.ShapeDtypeStruct((NC, NPAD, DH), jnp.float32),
        jax.ShapeDtypeStruct((NC, NPAD, DH), jnp.float32),
    ),
    mesh=plsc.VectorSubcoreMesh(core_axis_name="c", subcore_axis_name="s",
                                num_cores=NC, num_subcores=NS),
    compiler_params=pltpu.CompilerParams(use_tc_tiling_on_sc=False),
    scratch_types=[
        pltpu.VMEM_SHARED((NPAD, DH), jnp.float32),
        pltpu.VMEM((K, DH), jnp.float32),
        pltpu.VMEM((K, DH), jnp.float32),
        pltpu.VMEM((K, DH), jnp.float32),
        pltpu.VMEM((K, DH), jnp.float32),
        pltpu.VMEM((K, DH), jnp.float32),
        pltpu.VMEM((K, DH), jnp.float32),
        pltpu.VMEM((2, G, K), jnp.int32),
        pltpu.VMEM((2, G, K), jnp.int32),
        pltpu.VMEM((2, G, K), jnp.float32),
        pltpu.SemaphoreType.DMA,
        pltpu.SemaphoreType.DMA,
        pltpu.SemaphoreType.DMA,
        pltpu.SemaphoreType.DMA,
        pltpu.SemaphoreType.DMA,
        pltpu.SemaphoreType.DMA,
        pltpu.SemaphoreType.DMA,
        pltpu.SemaphoreType.DMA,
        pltpu.SemaphoreType.DMA,
        pltpu.SemaphoreType.DMA,
        pltpu.SemaphoreType.DMA,
        pltpu.SemaphoreType.DMA,
        pltpu.SemaphoreType.DMA,
        pltpu.SemaphoreType.DMA,
        pltpu.SemaphoreType.DMA,
    ],
)


def kernel(user_preference, item_preference, edge_values, edge_index):
    p0 = jnp.concatenate([user_preference, item_preference], axis=0)
    p0 = jnp.pad(p0, ((0, NPAD - N_NODES), (0, 0)))
    p = jnp.stack([p0[:, :DH], p0[:, DH:]])
    acc = p
    rows3 = edge_index[0].astype(jnp.int32).reshape(NS * NB, G, K)
    cols3 = edge_index[1].astype(jnp.int32).reshape(NS * NB, G, K)
    vals3 = edge_values.astype(jnp.float32).reshape(NS * NB, G, K)
    for _ in range(N_LAYERS):
        p, acc = _layer(p, acc, rows3, cols3, vals3)
    full = jnp.concatenate([acc[0], acc[1]], axis=1)[:N_NODES]
    return (full[:N_USERS], full[N_USERS:])


# speedup vs baseline: 1.1723x; 1.1436x over previous
import jax
import jax.numpy as jnp
from jax import lax
from jax.experimental import pallas as pl
from jax.experimental.pallas import tpu as pltpu
from jax.experimental.pallas import tpu_sc as plsc

N_USERS = 20000
N_ITEMS = 30000
N_NODES = N_USERS + N_ITEMS
N_EDGES = 800000
EMBED_DIM = 64
N_LAYERS = 3

NC = 2
NS = 16
L = 16

NPAD = 50176
DH = EMBED_DIM // NC

ZR = NPAD // NS
QR = 112
NQ = ZR // QR
NBUF = 4

K = 128
G = 17
BLK = K * G
NB = 23
E_TILE = BLK * NB
E_PAD = NS * E_TILE


def _layer_body(p_prev, acc_prev, rows3, cols3, vals3,
                p_next, acc_next,
                table,
                gbufa, gbufb, gbufc, gbufd, gbufe, gbuff, colv3, rowv3, valv3,
                gsa, gsb, gsc, gsd, gse, gsf,
                ssa, ssb, ssc, ssd, sse, ssf, ri, ci, vi):
    c = lax.axis_index("c")
    s = lax.axis_index("s")

    gbufs = (gbufa, gbufb, gbufc, gbufd, gbufe, gbuff)
    gsems = (gsa, gsb, gsc, gsd, gse, gsf)
    ssems = (ssa, ssb, ssc, ssd, sse, ssf)

    def _zfill(r, carry):
        for j in range(DH // L):
            gbufa[r, pl.ds(j * L, L)] = jnp.zeros((L,), jnp.float32)
        return carry
    lax.fori_loop(0, QR, _zfill, 0)
    zd = [pltpu.async_copy(gbufa.at[pl.ds(0, QR)],
                           table.at[pl.ds(s * ZR + q * QR, QR)], gsa)
          for q in range(NQ)]
    for d in zd:
        d.wait()
    plsc.subcore_barrier()

    psrc = p_prev.at[c]

    pltpu.async_copy(rows3.at[s * NB], rowv3.at[0], ri)
    pltpu.async_copy(cols3.at[s * NB], colv3.at[0], ci)
    pltpu.async_copy(vals3.at[s * NB], valv3.at[0], vi)

    def _block(b, carry):
        par = b % 2
        rowv2 = rowv3.at[par]
        colv2 = colv3.at[par]
        valv2 = valv3.at[par]
        pltpu.make_async_copy(rows3.at[s * NB], rowv2, ri).wait()
        pltpu.make_async_copy(cols3.at[s * NB], colv2, ci).wait()
        pltpu.make_async_copy(vals3.at[s * NB], valv2, vi).wait()

        @pl.when(b + 1 < NB)
        def _prefetch():
            nxt = (b + 1) % 2
            pltpu.async_copy(rows3.at[s * NB + b + 1], rowv3.at[nxt], ri)
            pltpu.async_copy(cols3.at[s * NB + b + 1], colv3.at[nxt], ci)
            pltpu.async_copy(vals3.at[s * NB + b + 1], valv3.at[nxt], vi)

        def _mul(g, buf):
            @plsc.parallel_loop(0, K // L)
            def _mq(q):
                vv16 = valv2[g, pl.ds(q * L, L)]
                for i in range(L):
                    vv = jnp.full((L,), vv16[i], jnp.float32)
                    for j in range(DH // L):
                        sl = (q * L + i, pl.ds(j * L, L))
                        buf[sl] = buf[sl] * vv

        gd = [None] * G
        sd = [None] * G
        for g in range(NBUF - 1):
            x = g % NBUF
            gd[g] = pltpu.async_copy(psrc.at[colv2.at[g]], gbufs[x], gsems[x])
        for gg in range(NBUF - 1, G + NBUF - 1):
            if gg < G:
                x = gg % NBUF
                if gg >= NBUF:
                    sd[gg - NBUF].wait()
                gd[gg] = pltpu.async_copy(psrc.at[colv2.at[gg]], gbufs[x],
                                          gsems[x])
            p = gg - (NBUF - 1)
            y = p % NBUF
            gd[p].wait()
            _mul(p, gbufs[y])
            sd[p] = pltpu.async_copy(gbufs[y], table.at[rowv2.at[p]],
                                     ssems[y], add=True)
        for p in range(G - NBUF, G):
            if p >= 0:
                sd[p].wait()
        return carry
    lax.fori_loop(0, NB, _block, 0)
    plsc.subcore_barrier()

    pd = pltpu.async_copy(table.at[pl.ds(s * ZR, ZR)],
                          p_next.at[c].at[pl.ds(s * ZR, ZR)], gse)
    tb = (gbufa, gbufb)
    ab = (gbufc, gbufd)
    st = (gsa, gsb)
    sa = (gsc, gsd)
    sw = (ssa, ssb)
    td = [None] * NQ
    ad = [None] * NQ
    wd = [None] * NQ
    for i in range(NQ + 1):
        x = i % 2
        if i < NQ:
            tro = s * ZR + i * QR
            if i >= 2:
                wd[i - 2].wait()
            td[i] = pltpu.async_copy(table.at[pl.ds(tro, QR)],
                                     tb[x].at[pl.ds(0, QR)], st[x])
            ad[i] = pltpu.async_copy(acc_prev.at[c].at[pl.ds(tro, QR)],
                                     ab[x].at[pl.ds(0, QR)], sa[x])
        if i >= 1:
            y = (i - 1) % 2
            td[i - 1].wait()
            ad[i - 1].wait()

            def _acc(r, carry):
                for j in range(DH // L):
                    tb[y][r, pl.ds(j * L, L)] = (tb[y][r, pl.ds(j * L, L)]
                                                 + ab[y][r, pl.ds(j * L, L)])
                return carry
            lax.fori_loop(0, QR, _acc, 0)
            wd[i - 1] = pltpu.async_copy(
                tb[y].at[pl.ds(0, QR)],
                acc_next.at[c].at[pl.ds(s * ZR + (i - 1) * QR, QR)],
                sw[y])
    wd[NQ - 2].wait()
    wd[NQ - 1].wait()
    pd.wait()


_layer = pl.kernel(
    _layer_body,
    out_type=(
        jax.ShapeDtypeStruct((NC, NPAD, DH), jnp.float32),
        jax.ShapeDtypeStruct((NC, NPAD, DH), jnp.float32),
    ),
    mesh=plsc.VectorSubcoreMesh(core_axis_name="c", subcore_axis_name="s",
                                num_cores=NC, num_subcores=NS),
    compiler_params=pltpu.CompilerParams(use_tc_tiling_on_sc=False),
    scratch_types=[
        pltpu.VMEM_SHARED((NPAD, DH), jnp.float32),
        pltpu.VMEM((K, DH), jnp.float32),
        pltpu.VMEM((K, DH), jnp.float32),
        pltpu.VMEM((K, DH), jnp.float32),
        pltpu.VMEM((K, DH), jnp.float32),
        pltpu.VMEM((K, DH), jnp.float32),
        pltpu.VMEM((K, DH), jnp.float32),
        pltpu.VMEM((2, G, K), jnp.int32),
        pltpu.VMEM((2, G, K), jnp.int32),
        pltpu.VMEM((2, G, K), jnp.float32),
        pltpu.SemaphoreType.DMA,
        pltpu.SemaphoreType.DMA,
        pltpu.SemaphoreType.DMA,
        pltpu.SemaphoreType.DMA,
        pltpu.SemaphoreType.DMA,
        pltpu.SemaphoreType.DMA,
        pltpu.SemaphoreType.DMA,
        pltpu.SemaphoreType.DMA,
        pltpu.SemaphoreType.DMA,
        pltpu.SemaphoreType.DMA,
        pltpu.SemaphoreType.DMA,
        pltpu.SemaphoreType.DMA,
        pltpu.SemaphoreType.DMA,
        pltpu.SemaphoreType.DMA,
        pltpu.SemaphoreType.DMA,
    ],
)


def kernel(user_preference, item_preference, edge_values, edge_index):
    p0 = jnp.concatenate([user_preference, item_preference], axis=0)
    p0 = jnp.pad(p0, ((0, NPAD - N_NODES), (0, 0)))
    p = jnp.stack([p0[:, :DH], p0[:, DH:]])
    acc = p
    epad = E_PAD - N_EDGES
    rows3 = jnp.pad(edge_index[0].astype(jnp.int32),
                    (0, epad)).reshape(NS * NB, G, K)
    cols3 = jnp.pad(edge_index[1].astype(jnp.int32),
                    (0, epad)).reshape(NS * NB, G, K)
    vals3 = jnp.pad(edge_values.astype(jnp.float32),
                    (0, epad)).reshape(NS * NB, G, K)
    for _ in range(N_LAYERS):
        p, acc = _layer(p, acc, rows3, cols3, vals3)
    full = jnp.concatenate([acc[0], acc[1]], axis=1)[:N_NODES]
    return (full[:N_USERS], full[N_USERS:])
